# baseline (speedup 1.0000x reference)
import jax
import jax.numpy as jnp
from jax.experimental import pallas as pl
from jax.experimental.pallas import tpu as pltpu


def _bf(x):
    return x


def _embed_body(idx_ref, tok_ref, pos_ref, o_ref):
    o_ref[...] = tok_ref[...] + pos_ref[...]


def _embed(idx_flat, tok_emb, pos_emb):
    T = idx_flat.shape[0]
    V, D = tok_emb.shape
    out = pl.pallas_call(
        _embed_body,
        grid_spec=pltpu.PrefetchScalarGridSpec(
            num_scalar_prefetch=1,
            grid=(T,),
            in_specs=[
                pl.BlockSpec((1, 1, D), lambda i, idx: (idx[i], 0, 0)),
                pl.BlockSpec((1, 1, D), lambda i, idx: (i, 0, 0)),
            ],
            out_specs=pl.BlockSpec((1, 1, D), lambda i, idx: (i, 0, 0)),
        ),
        out_shape=jax.ShapeDtypeStruct((T, 1, D), jnp.float32),
    )(idx_flat, tok_emb.reshape(V, 1, D), pos_emb.reshape(T, 1, D))
    return out.reshape(T, D)


def _ln_body(x_ref, g_ref, b_ref, o_ref):
    x = x_ref[...]
    mu = jnp.mean(x, axis=1, keepdims=True)
    var = jnp.mean((x - mu) ** 2, axis=1, keepdims=True)
    o_ref[...] = (x - mu) * jax.lax.rsqrt(var + 1e-5) * g_ref[...] + b_ref[...]


def _layer_norm(x, g, b):
    T, D = x.shape
    return pl.pallas_call(
        _ln_body,
        out_shape=jax.ShapeDtypeStruct((T, D), jnp.float32),
    )(x, g.reshape(1, D), b.reshape(1, D))


def _attn_body(h_ref, wq_ref, wk_ref, wv_ref, o_ref, *, scale):
    h = _bf(h_ref[...])
    q = jnp.dot(h, _bf(wq_ref[0]), preferred_element_type=jnp.float32)
    k = jnp.dot(h, _bf(wk_ref[0]), preferred_element_type=jnp.float32)
    v = jnp.dot(h, _bf(wv_ref[0]), preferred_element_type=jnp.float32)
    s = jnp.dot(_bf(q), _bf(k).T, preferred_element_type=jnp.float32) * scale
    T = s.shape[0]
    ri = jax.lax.broadcasted_iota(jnp.int32, (T, T), 0)
    ci = jax.lax.broadcasted_iota(jnp.int32, (T, T), 1)
    s = jnp.where(ci <= ri, s, -jnp.inf)
    m = jnp.max(s, axis=1, keepdims=True)
    p = jnp.exp(s - m)
    p = p / jnp.sum(p, axis=1, keepdims=True)
    o_ref[0] = jnp.dot(_bf(p), _bf(v), preferred_element_type=jnp.float32)


def _attention(h, Wq, Wk, Wv, scale):
    T, D = h.shape
    H, _, HS = Wq.shape
    import functools
    return pl.pallas_call(
        functools.partial(_attn_body, scale=scale),
        grid=(H,),
        in_specs=[
            pl.BlockSpec((T, D), lambda hh: (0, 0)),
            pl.BlockSpec((1, D, HS), lambda hh: (hh, 0, 0)),
            pl.BlockSpec((1, D, HS), lambda hh: (hh, 0, 0)),
            pl.BlockSpec((1, D, HS), lambda hh: (hh, 0, 0)),
        ],
        out_specs=pl.BlockSpec((1, T, HS), lambda hh: (hh, 0, 0)),
        out_shape=jax.ShapeDtypeStruct((H, T, HS), jnp.float32),
    )(h, Wq, Wk, Wv)


def _proj_body(att_ref, wp_ref, x_ref, bp_ref, o_ref):
    hh = pl.program_id(0)

    @pl.when(hh == 0)
    def _():
        o_ref[...] = x_ref[...] + bp_ref[...]

    o_ref[...] += jnp.dot(_bf(att_ref[0]), _bf(wp_ref[0]),
                          preferred_element_type=jnp.float32)


def _proj_residual(att, Wproj_h, x, bproj):
    H, T, HS = att.shape
    D = x.shape[1]
    return pl.pallas_call(
        _proj_body,
        grid=(H,),
        in_specs=[
            pl.BlockSpec((1, T, HS), lambda hh: (hh, 0, 0)),
            pl.BlockSpec((1, HS, D), lambda hh: (hh, 0, 0)),
            pl.BlockSpec((T, D), lambda hh: (0, 0)),
            pl.BlockSpec((1, D), lambda hh: (0, 0)),
        ],
        out_specs=pl.BlockSpec((T, D), lambda hh: (0, 0)),
        out_shape=jax.ShapeDtypeStruct((T, D), jnp.float32),
    )(att, Wproj_h, x, bproj.reshape(1, D))


def _gate_body(h2_ref, wg_ref, o_ref):
    o_ref[...] = jnp.dot(_bf(h2_ref[...]), _bf(wg_ref[...]),
                         preferred_element_type=jnp.float32)


def _gate(h2, Wg):
    T, D = h2.shape
    E = Wg.shape[1]
    return pl.pallas_call(
        _gate_body,
        out_shape=jax.ShapeDtypeStruct((T, E), jnp.float32),
    )(h2, Wg)


def _moe_body(gl_ref, h2_ref, w1_ref, b1_ref, w2_ref, b2_ref, x2_ref,
              o_ref, acc_ref):
    e = pl.program_id(0)
    j = pl.program_id(1)
    nj = pl.num_programs(1)

    @pl.when((e == 0) & (j == 0))
    def _():
        o_ref[...] = x2_ref[...]

    @pl.when(j == 0)
    def _():
        acc_ref[...] = jnp.zeros_like(acc_ref)

    t = jnp.dot(_bf(h2_ref[...]), _bf(w1_ref[0]),
                preferred_element_type=jnp.float32) + b1_ref[0, 0]
    t = jnp.maximum(t, 0.0)
    acc_ref[...] += jnp.dot(_bf(t), _bf(w2_ref[0]),
                            preferred_element_type=jnp.float32)

    @pl.when(j == nj - 1)
    def _():
        gl = gl_ref[...]
        sel = jnp.argmax(gl, axis=1)
        mask = (sel == e)[:, None]
        o_ref[...] += jnp.where(mask, acc_ref[...] + b2_ref[0, 0], 0.0)


def _moe(gl, h2, W1, b1, W2, b2, x2):
    T, D = h2.shape
    E, _, DFF = W1.shape
    DBLK = 1024
    nj = DFF // DBLK
    return pl.pallas_call(
        _moe_body,
        grid=(E, nj),
        in_specs=[
            pl.BlockSpec((T, gl.shape[1]), lambda e, j: (0, 0)),
            pl.BlockSpec((T, D), lambda e, j: (0, 0)),
            pl.BlockSpec((1, D, DBLK), lambda e, j: (e, 0, j)),
            pl.BlockSpec((1, 1, DBLK), lambda e, j: (e, 0, j)),
            pl.BlockSpec((1, DBLK, D), lambda e, j: (e, j, 0)),
            pl.BlockSpec((1, 1, D), lambda e, j: (e, 0, 0)),
            pl.BlockSpec((T, D), lambda e, j: (0, 0)),
        ],
        out_specs=pl.BlockSpec((T, D), lambda e, j: (0, 0)),
        out_shape=jax.ShapeDtypeStruct((T, D), jnp.float32),
        scratch_shapes=[pltpu.VMEM((T, D), jnp.float32)],
    )(gl, h2, W1.astype(jnp.float32), b1.reshape(E, 1, DFF),
      W2.astype(jnp.float32), b2.reshape(E, 1, D), x2)


def _lm_body(x_ref, w_ref, b_ref, o_ref):
    o_ref[0] = jnp.dot(_bf(x_ref[...]), _bf(w_ref[...]),
                       preferred_element_type=jnp.float32) + b_ref[...]


def _lm_head(x, Wlm, blm):
    T, D = x.shape
    V = Wlm.shape[1]
    NBLK = 1280
    return pl.pallas_call(
        _lm_body,
        grid=(V // NBLK,),
        in_specs=[
            pl.BlockSpec((T, D), lambda n: (0, 0)),
            pl.BlockSpec((D, NBLK), lambda n: (0, n)),
            pl.BlockSpec((1, NBLK), lambda n: (0, n)),
        ],
        out_specs=pl.BlockSpec((1, T, NBLK), lambda n: (0, 0, n)),
        out_shape=jax.ShapeDtypeStruct((1, T, V), jnp.float32),
    )(x, Wlm, blm.reshape(1, V))


def kernel(idx, tok_emb, pos_emb, ln1_g, ln1_b, Wq, Wk, Wv, Wproj, bproj,
           ln2_g, ln2_b, Wg, W1, b1, W2, b2, Wlm, blm):
    Bx, T = idx.shape
    D = tok_emb.shape[1]
    H, _, HS = Wq.shape

    x = _embed(idx.reshape(T).astype(jnp.int32), tok_emb, pos_emb)
    h = _layer_norm(x, ln1_g, ln1_b)
    att = _attention(h, Wq, Wk, Wv, float(D) ** -0.5)
    x2 = _proj_residual(att, Wproj.reshape(H, HS, D), x, bproj)
    h2 = _layer_norm(x2, ln2_g, ln2_b)
    gl = _gate(h2, Wg)
    x3 = _moe(gl, h2, W1, b1, W2, b2, x2)
    logits = _lm_head(x3, Wlm, blm)
    return logits.reshape(Bx, T, -1)
